# Initial kernel scaffold; baseline (speedup 1.0000x reference)
import jax
import jax.numpy as jnp
from jax.experimental import pallas as pl


def kernel(center_coords, coords, types, target):
    raise NotImplementedError("write your pallas kernel here")


# speedup vs baseline: 8.1633x; 8.1633x over previous
import functools

import jax
import jax.numpy as jnp
from jax import lax
from jax.experimental import pallas as pl
from jax.experimental.pallas import tpu as pltpu
from jax.experimental.pallas import tpu_sc as plsc

K = 10


def _topk_body(c_ref, t_ref, o_ref, *, n_target):
    b = pl.program_id(0)
    cc = c_ref[0]
    tt = t_ref[0]
    d = (cc[:, 0:1] - tt[0:1, :]) ** 2
    d = d + (cc[:, 1:2] - tt[1:2, :]) ** 2
    d = d + (cc[:, 2:3] - tt[2:3, :]) ** 2
    iota = lax.broadcasted_iota(jnp.int32, d.shape, 1)
    base = b * n_target
    for k in range(K):
        m = jnp.min(d, axis=1, keepdims=True)
        am = jnp.min(jnp.where(d == m, iota, n_target), axis=1)
        o_ref[0, :, k : k + 1] = (am + base)[:, None]
        d = jnp.where(iota == am[:, None], jnp.inf, d)


def _topk_indices(center_coords, target_t):
    B, C, _ = center_coords.shape
    T = target_t.shape[-1]
    CB = 256
    nb = C // CB
    return pl.pallas_call(
        functools.partial(_topk_body, n_target=T),
        grid=(B, nb),
        in_specs=[
            pl.BlockSpec((1, CB, 3), lambda b, i: (b, i, 0)),
            pl.BlockSpec((1, 3, T), lambda b, i: (b, 0, 0)),
        ],
        out_specs=pl.BlockSpec((1, CB, K), lambda b, i: (b, i, 0)),
        out_shape=jax.ShapeDtypeStruct((B, C, K), jnp.int32),
    )(center_coords, target_t)


def _sc_gather(table, idx):
    N = idx.shape[0]
    info = plsc.get_sparse_core_info()
    nw = info.num_cores * info.num_subcores
    n_per = N // nw
    CH = 128
    nch = n_per // CH
    mesh = plsc.VectorSubcoreMesh(core_axis_name="c", subcore_axis_name="s")

    @functools.partial(
        pl.kernel,
        mesh=mesh,
        compiler_params=pltpu.CompilerParams(use_tc_tiling_on_sc=False),
        out_type=jax.ShapeDtypeStruct((N, 16), jnp.float32),
        scratch_types=[
            pltpu.VMEM((n_per,), jnp.int32),
            pltpu.VMEM((n_per, 16), jnp.float32),
            pltpu.SemaphoreType.DMA,
        ],
    )
    def gather_k(table_hbm, idx_hbm, out_hbm, idx_v, rows_v, sem):
        wid = lax.axis_index("s") * info.num_cores + lax.axis_index("c")
        base = wid * n_per
        pltpu.sync_copy(idx_hbm.at[pl.ds(base, n_per)], idx_v)
        copies = []
        for j in range(nch):
            copies.append(
                pltpu.async_copy(
                    table_hbm.at[idx_v.at[pl.ds(j * CH, CH)]],
                    rows_v.at[pl.ds(j * CH, CH)],
                    sem,
                )
            )
        for c in copies:
            c.wait()
        pltpu.sync_copy(rows_v, out_hbm.at[pl.ds(base, n_per)])

    return gather_k(table, idx)


def _local_body(gx_ref, gy_ref, gz_ref, tyT_ref, sx_ref, sy_ref, sz_ref, st_ref,
                o_ref, *, G, TN):
    gx = gx_ref[0]
    gy = gy_ref[0]
    gz = gz_ref[0]
    sx = sx_ref[0]
    sy = sy_ref[0]
    sz = sz_ref[0]
    sti = st_ref[0].astype(jnp.int32)
    iota_g = lax.broadcasted_iota(jnp.int32, gx.shape, 1)

    M = [tyT_ref[0, :, t, :] for t in range(TN)]
    mx = M[0]
    amax_t = jnp.zeros_like(iota_g)
    for t in range(1, TN):
        hi = M[t] > mx
        amax_t = jnp.where(hi, t, amax_t)
        mx = jnp.where(hi, M[t], mx)

    demd_gen = jnp.full(gx.shape, 3.4e38, jnp.float32)
    ks = jnp.zeros_like(iota_g)
    s_emd_tgt = jnp.float32(0.0)
    s_picked_tgt = jnp.float32(0.0)
    s_corr_tgt = jnp.float32(0.0)
    for k in range(K):
        dk = (gx - sx[:, k : k + 1]) ** 2
        dk = dk + (gy - sy[:, k : k + 1]) ** 2
        dk = dk + (gz - sz[:, k : k + 1]) ** 2
        lo = dk < demd_gen
        ks = jnp.where(lo, k, ks)
        demd_gen = jnp.where(lo, dk, demd_gen)
        mg = jnp.min(dk, axis=1, keepdims=True)
        s_emd_tgt = s_emd_tgt + jnp.sum(mg)
        gs_k = jnp.min(jnp.where(dk == mg, iota_g, G), axis=1, keepdims=True)
        ohg = iota_g == gs_k
        ttk = sti[:, k : k + 1]
        pick_cg = jnp.zeros_like(gx)
        for t in range(TN):
            pick_cg = pick_cg + jnp.where(ttk == t, M[t], 0.0)
        s_picked_tgt = s_picked_tgt + jnp.sum(jnp.where(ohg, pick_cg, 0.0))
        am_sel = jnp.sum(jnp.where(ohg, amax_t, 0), axis=1, keepdims=True)
        s_corr_tgt = s_corr_tgt + jnp.sum((am_sel == ttk).astype(jnp.float32))

    s_emd_gen = jnp.sum(demd_gen)
    tt_gen = jnp.zeros_like(iota_g)
    for k in range(K):
        tt_gen = tt_gen + jnp.where(ks == k, sti[:, k : k + 1], 0)
    picked_gen = jnp.zeros_like(gx)
    for t in range(TN):
        picked_gen = picked_gen + jnp.where(tt_gen == t, M[t], 0.0)
    s_picked_gen = jnp.sum(picked_gen)
    s_corr_gen = jnp.sum((amax_t == tt_gen).astype(jnp.float32))

    lane = lax.broadcasted_iota(jnp.int32, (1, 128), 1)
    vec = jnp.where(lane == 0, s_emd_gen, 0.0)
    vec = vec + jnp.where(lane == 1, s_emd_tgt, 0.0)
    vec = vec + jnp.where(lane == 2, s_picked_gen, 0.0)
    vec = vec + jnp.where(lane == 3, s_corr_gen, 0.0)
    vec = vec + jnp.where(lane == 4, s_picked_tgt, 0.0)
    vec = vec + jnp.where(lane == 5, s_corr_tgt, 0.0)
    o_ref[0] = vec


def _local_losses(gx, gy, gz, types_t, sx, sy, sz, st):
    B, C, G = gx.shape
    TN = types_t.shape[2]
    CB = 256
    nb = C // CB
    spec_g = pl.BlockSpec((1, CB, G), lambda b, i: (b, i, 0))
    spec_k = pl.BlockSpec((1, CB, K), lambda b, i: (b, i, 0))
    out = pl.pallas_call(
        functools.partial(_local_body, G=G, TN=TN),
        grid=(B, nb),
        in_specs=[
            spec_g, spec_g, spec_g,
            pl.BlockSpec((1, CB, TN, G), lambda b, i: (b, i, 0, 0)),
            spec_k, spec_k, spec_k, spec_k,
        ],
        out_specs=pl.BlockSpec((1, 1, 128), lambda b, i: (b * nb + i, 0, 0)),
        out_shape=jax.ShapeDtypeStruct((B * nb, 1, 128), jnp.float32),
    )(gx, gy, gz, types_t, sx, sy, sz, st)
    return jnp.sum(out, axis=(0, 1))


def kernel(center_coords, coords, types, target):
    B, C, _ = center_coords.shape
    G = coords.shape[2]
    T = target.shape[1]

    target_t = jnp.transpose(target, (0, 2, 1))[:, :3]
    idx = _topk_indices(center_coords, target_t)

    table = jnp.pad(target, ((0, 0), (0, 0), (0, 12))).reshape(B * T, 16)
    rows = _sc_gather(table, idx.reshape(-1))
    ge = rows.reshape(B, C, K, 16)
    sx, sy, sz, st = ge[..., 0], ge[..., 1], ge[..., 2], ge[..., 3]

    gx, gy, gz = coords[..., 0], coords[..., 1], coords[..., 2]
    types_t = jnp.transpose(types, (0, 1, 3, 2))
    p = _local_losses(gx, gy, gz, types_t, sx, sy, sz, st)

    n_g = B * C * G
    n_k = B * C * K
    gen_type_loss = -p[2] / n_g
    target_type_loss = -p[4] / n_k
    emd_loss = p[0] / n_g + p[1] / n_k
    gen_type_correct = p[3] / n_g
    target_type_correct = p[5] / n_k
    loss = gen_type_loss + target_type_loss + emd_loss
    return (loss, gen_type_loss, target_type_loss, emd_loss,
            gen_type_correct, target_type_correct)


# speedup vs baseline: 9.0452x; 1.1080x over previous
import functools

import jax
import jax.numpy as jnp
from jax import lax
from jax.experimental import pallas as pl
from jax.experimental.pallas import tpu as pltpu
from jax.experimental.pallas import tpu_sc as plsc

K = 10


def _topk_body(c_ref, t_ref, o_ref, *, n_target):
    b = pl.program_id(0)
    cc = c_ref[0]
    tt = t_ref[0]
    d = (cc[:, 0:1] - tt[0:1, :]) ** 2
    d = d + (cc[:, 1:2] - tt[1:2, :]) ** 2
    d = d + (cc[:, 2:3] - tt[2:3, :]) ** 2
    iota = lax.broadcasted_iota(jnp.int32, d.shape, 1)
    key = (lax.bitcast_convert_type(d, jnp.int32) & jnp.int32(-4096)) | iota
    base = b * n_target
    for k in range(K):
        m = jnp.min(key, axis=1, keepdims=True)
        o_ref[0, :, k : k + 1] = (m & 4095) + base
        key = jnp.where(key == m, jnp.int32(0x7FFFFFFF), key)


def _topk_indices(center_coords, target_t):
    B, C, _ = center_coords.shape
    T = target_t.shape[-1]
    CB = 256
    nb = C // CB
    return pl.pallas_call(
        functools.partial(_topk_body, n_target=T),
        grid=(B, nb),
        in_specs=[
            pl.BlockSpec((1, CB, 3), lambda b, i: (b, i, 0)),
            pl.BlockSpec((1, 3, T), lambda b, i: (b, 0, 0)),
        ],
        out_specs=pl.BlockSpec((1, CB, K), lambda b, i: (b, i, 0)),
        out_shape=jax.ShapeDtypeStruct((B, C, K), jnp.int32),
    )(center_coords, target_t)


def _sc_gather(table, idx):
    N = idx.shape[0]
    info = plsc.get_sparse_core_info()
    nw = info.num_cores * info.num_subcores
    n_per = N // nw
    CH = 128
    nch = n_per // CH
    mesh = plsc.VectorSubcoreMesh(core_axis_name="c", subcore_axis_name="s")

    @functools.partial(
        pl.kernel,
        mesh=mesh,
        compiler_params=pltpu.CompilerParams(use_tc_tiling_on_sc=False),
        out_type=jax.ShapeDtypeStruct((N, 16), jnp.float32),
        scratch_types=[
            pltpu.VMEM((n_per,), jnp.int32),
            pltpu.VMEM((n_per, 16), jnp.float32),
            pltpu.SemaphoreType.DMA,
        ],
    )
    def gather_k(table_hbm, idx_hbm, out_hbm, idx_v, rows_v, sem):
        wid = lax.axis_index("s") * info.num_cores + lax.axis_index("c")
        base = wid * n_per
        pltpu.sync_copy(idx_hbm.at[pl.ds(base, n_per)], idx_v)
        copies = []
        for j in range(nch):
            copies.append(
                pltpu.async_copy(
                    table_hbm.at[idx_v.at[pl.ds(j * CH, CH)]],
                    rows_v.at[pl.ds(j * CH, CH)],
                    sem,
                )
            )
        for c in copies:
            c.wait()
        pltpu.sync_copy(rows_v, out_hbm.at[pl.ds(base, n_per)])

    return gather_k(table, idx)


def _local_body(gx_ref, gy_ref, gz_ref, tyT_ref, sx_ref, sy_ref, sz_ref, st_ref,
                o_ref, *, G, TN):
    gx = gx_ref[0]
    gy = gy_ref[0]
    gz = gz_ref[0]
    sx = sx_ref[0]
    sy = sy_ref[0]
    sz = sz_ref[0]
    sti = st_ref[0].astype(jnp.int32)
    iota_g = lax.broadcasted_iota(jnp.int32, gx.shape, 1)

    M = [tyT_ref[0, :, t, :] for t in range(TN)]
    mx = M[0]
    amax_t = jnp.zeros_like(iota_g)
    for t in range(1, TN):
        hi = M[t] > mx
        amax_t = jnp.where(hi, t, amax_t)
        mx = jnp.where(hi, M[t], mx)

    demd_gen = jnp.full(gx.shape, 3.4e38, jnp.float32)
    ks = jnp.zeros_like(iota_g)
    s_emd_tgt = jnp.float32(0.0)
    s_picked_tgt = jnp.float32(0.0)
    s_corr_tgt = jnp.float32(0.0)
    for k in range(K):
        dk = (gx - sx[:, k : k + 1]) ** 2
        dk = dk + (gy - sy[:, k : k + 1]) ** 2
        dk = dk + (gz - sz[:, k : k + 1]) ** 2
        lo = dk < demd_gen
        ks = jnp.where(lo, k, ks)
        demd_gen = jnp.where(lo, dk, demd_gen)
        mg = jnp.min(dk, axis=1, keepdims=True)
        s_emd_tgt = s_emd_tgt + jnp.sum(mg)
        gs_k = jnp.min(jnp.where(dk == mg, iota_g, G), axis=1, keepdims=True)
        ohg = iota_g == gs_k
        ttk = sti[:, k : k + 1]
        pick_cg = jnp.zeros_like(gx)
        for t in range(TN):
            pick_cg = pick_cg + jnp.where(ttk == t, M[t], 0.0)
        s_picked_tgt = s_picked_tgt + jnp.sum(jnp.where(ohg, pick_cg, 0.0))
        am_sel = jnp.sum(jnp.where(ohg, amax_t, 0), axis=1, keepdims=True)
        s_corr_tgt = s_corr_tgt + jnp.sum((am_sel == ttk).astype(jnp.float32))

    s_emd_gen = jnp.sum(demd_gen)
    tt_gen = jnp.zeros_like(iota_g)
    for k in range(K):
        tt_gen = tt_gen + jnp.where(ks == k, sti[:, k : k + 1], 0)
    picked_gen = jnp.zeros_like(gx)
    for t in range(TN):
        picked_gen = picked_gen + jnp.where(tt_gen == t, M[t], 0.0)
    s_picked_gen = jnp.sum(picked_gen)
    s_corr_gen = jnp.sum((amax_t == tt_gen).astype(jnp.float32))

    lane = lax.broadcasted_iota(jnp.int32, (1, 128), 1)
    vec = jnp.where(lane == 0, s_emd_gen, 0.0)
    vec = vec + jnp.where(lane == 1, s_emd_tgt, 0.0)
    vec = vec + jnp.where(lane == 2, s_picked_gen, 0.0)
    vec = vec + jnp.where(lane == 3, s_corr_gen, 0.0)
    vec = vec + jnp.where(lane == 4, s_picked_tgt, 0.0)
    vec = vec + jnp.where(lane == 5, s_corr_tgt, 0.0)
    o_ref[0] = vec


def _local_losses(gx, gy, gz, types_t, sx, sy, sz, st):
    B, C, G = gx.shape
    TN = types_t.shape[2]
    CB = 256
    nb = C // CB
    spec_g = pl.BlockSpec((1, CB, G), lambda b, i: (b, i, 0))
    spec_k = pl.BlockSpec((1, CB, K), lambda b, i: (b, i, 0))
    out = pl.pallas_call(
        functools.partial(_local_body, G=G, TN=TN),
        grid=(B, nb),
        in_specs=[
            spec_g, spec_g, spec_g,
            pl.BlockSpec((1, CB, TN, G), lambda b, i: (b, i, 0, 0)),
            spec_k, spec_k, spec_k, spec_k,
        ],
        out_specs=pl.BlockSpec((1, 1, 128), lambda b, i: (b * nb + i, 0, 0)),
        out_shape=jax.ShapeDtypeStruct((B * nb, 1, 128), jnp.float32),
    )(gx, gy, gz, types_t, sx, sy, sz, st)
    return jnp.sum(out, axis=(0, 1))


def kernel(center_coords, coords, types, target):
    B, C, _ = center_coords.shape
    G = coords.shape[2]
    T = target.shape[1]

    target_t = jnp.transpose(target, (0, 2, 1))[:, :3]
    idx = _topk_indices(center_coords, target_t)

    table = jnp.pad(target, ((0, 0), (0, 0), (0, 12))).reshape(B * T, 16)
    rows = _sc_gather(table, idx.reshape(-1))
    ge = rows.reshape(B, C, K, 16)
    sx, sy, sz, st = ge[..., 0], ge[..., 1], ge[..., 2], ge[..., 3]

    gx, gy, gz = coords[..., 0], coords[..., 1], coords[..., 2]
    types_t = jnp.transpose(types, (0, 1, 3, 2))
    p = _local_losses(gx, gy, gz, types_t, sx, sy, sz, st)

    n_g = B * C * G
    n_k = B * C * K
    gen_type_loss = -p[2] / n_g
    target_type_loss = -p[4] / n_k
    emd_loss = p[0] / n_g + p[1] / n_k
    gen_type_correct = p[3] / n_g
    target_type_correct = p[5] / n_k
    loss = gen_type_loss + target_type_loss + emd_loss
    return (loss, gen_type_loss, target_type_loss, emd_loss,
            gen_type_correct, target_type_correct)


# speedup vs baseline: 21.2952x; 2.3543x over previous
import functools

import jax
import jax.numpy as jnp
from jax import lax
from jax.experimental import pallas as pl
from jax.experimental.pallas import tpu as pltpu
from jax.experimental.pallas import tpu_sc as plsc

K = 10


def _topk_body(c_ref, t_ref, o_ref, *, n_target):
    b = pl.program_id(0)
    cc = c_ref[0]
    tt = t_ref[0]
    d = (cc[:, 0:1] - tt[0:1, :]) ** 2
    d = d + (cc[:, 1:2] - tt[1:2, :]) ** 2
    d = d + (cc[:, 2:3] - tt[2:3, :]) ** 2
    iota = lax.broadcasted_iota(jnp.int32, d.shape, 1)
    key = (lax.bitcast_convert_type(d, jnp.int32) & jnp.int32(-4096)) | iota
    base = b * n_target
    for k in range(K):
        m = jnp.min(key, axis=1, keepdims=True)
        o_ref[0, :, k : k + 1] = (m & 4095) + base
        key = jnp.where(key == m, jnp.int32(0x7FFFFFFF), key)


def _topk_indices(center_coords, target_t):
    B, C, _ = center_coords.shape
    T = target_t.shape[-1]
    CB = 256
    nb = C // CB
    return pl.pallas_call(
        functools.partial(_topk_body, n_target=T),
        grid=(B, nb),
        in_specs=[
            pl.BlockSpec((1, CB, 3), lambda b, i: (b, i, 0)),
            pl.BlockSpec((1, 3, T), lambda b, i: (b, 0, 0)),
        ],
        out_specs=pl.BlockSpec((1, CB, K), lambda b, i: (b, i, 0)),
        out_shape=jax.ShapeDtypeStruct((B, C, K), jnp.int32),
    )(center_coords, target_t)


def _sc_gather(table, idx):
    N = idx.shape[0]
    info = plsc.get_sparse_core_info()
    nw = info.num_cores * info.num_subcores
    n_per = N // nw
    CH = 128
    nch = n_per // CH
    mesh = plsc.VectorSubcoreMesh(core_axis_name="c", subcore_axis_name="s")

    @functools.partial(
        pl.kernel,
        mesh=mesh,
        compiler_params=pltpu.CompilerParams(use_tc_tiling_on_sc=False),
        out_type=jax.ShapeDtypeStruct((N, 16), jnp.float32),
        scratch_types=[
            pltpu.VMEM((n_per,), jnp.int32),
            pltpu.VMEM((n_per, 16), jnp.float32),
            pltpu.SemaphoreType.DMA,
        ],
    )
    def gather_k(table_hbm, idx_hbm, out_hbm, idx_v, rows_v, sem):
        wid = lax.axis_index("s") * info.num_cores + lax.axis_index("c")
        base = wid * n_per
        pltpu.sync_copy(idx_hbm.at[pl.ds(base, n_per)], idx_v)
        copies = []
        for j in range(nch):
            copies.append(
                pltpu.async_copy(
                    table_hbm.at[idx_v.at[pl.ds(j * CH, CH)]],
                    rows_v.at[pl.ds(j * CH, CH)],
                    sem,
                )
            )
        for c in copies:
            c.wait()
        pltpu.sync_copy(rows_v, out_hbm.at[pl.ds(base, n_per)])

    return gather_k(table, idx)


def _local_body(gx_ref, gy_ref, gz_ref, tyT_ref, sx_ref, sy_ref, sz_ref, st_ref,
                o_ref, *, G, TN):
    gx = gx_ref[0]
    gy = gy_ref[0]
    gz = gz_ref[0]
    sx = sx_ref[0]
    sy = sy_ref[0]
    sz = sz_ref[0]
    sti = st_ref[0].astype(jnp.int32)
    iota_g = lax.broadcasted_iota(jnp.int32, gx.shape, 0)

    M = [tyT_ref[0, t] for t in range(TN)]
    mx = M[0]
    amax_t = jnp.zeros_like(iota_g)
    for t in range(1, TN):
        hi = M[t] > mx
        amax_t = jnp.where(hi, t, amax_t)
        mx = jnp.where(hi, M[t], mx)

    demd_gen = jnp.full(gx.shape, 3.4e38, jnp.float32)
    ks = jnp.zeros_like(iota_g)
    s_emd_tgt = jnp.float32(0.0)
    s_picked_tgt = jnp.float32(0.0)
    s_corr_tgt = jnp.float32(0.0)
    for k in range(K):
        dk = (gx - sx[k : k + 1]) ** 2
        dk = dk + (gy - sy[k : k + 1]) ** 2
        dk = dk + (gz - sz[k : k + 1]) ** 2
        lo = dk < demd_gen
        ks = jnp.where(lo, k, ks)
        demd_gen = jnp.where(lo, dk, demd_gen)
        mg = jnp.min(dk, axis=0, keepdims=True)
        s_emd_tgt = s_emd_tgt + jnp.sum(mg)
        gs_k = jnp.min(jnp.where(dk == mg, iota_g, G), axis=0, keepdims=True)
        ohg = iota_g == gs_k
        ttk = sti[k : k + 1]
        pick_cg = jnp.zeros_like(gx)
        for t in range(TN):
            pick_cg = pick_cg + jnp.where(ttk == t, M[t], 0.0)
        s_picked_tgt = s_picked_tgt + jnp.sum(jnp.where(ohg, pick_cg, 0.0))
        am_sel = jnp.sum(jnp.where(ohg, amax_t, 0), axis=0, keepdims=True)
        s_corr_tgt = s_corr_tgt + jnp.sum((am_sel == ttk).astype(jnp.float32))

    s_emd_gen = jnp.sum(demd_gen)
    tt_gen = jnp.zeros_like(iota_g)
    for k in range(K):
        tt_gen = tt_gen + jnp.where(ks == k, sti[k : k + 1], 0)
    picked_gen = jnp.zeros_like(gx)
    for t in range(TN):
        picked_gen = picked_gen + jnp.where(tt_gen == t, M[t], 0.0)
    s_picked_gen = jnp.sum(picked_gen)
    s_corr_gen = jnp.sum((amax_t == tt_gen).astype(jnp.float32))

    lane = lax.broadcasted_iota(jnp.int32, (1, 128), 1)
    vec = jnp.where(lane == 0, s_emd_gen, 0.0)
    vec = vec + jnp.where(lane == 1, s_emd_tgt, 0.0)
    vec = vec + jnp.where(lane == 2, s_picked_gen, 0.0)
    vec = vec + jnp.where(lane == 3, s_corr_gen, 0.0)
    vec = vec + jnp.where(lane == 4, s_picked_tgt, 0.0)
    vec = vec + jnp.where(lane == 5, s_corr_tgt, 0.0)
    o_ref[0] = vec


def _local_losses(gx, gy, gz, types_t, sx, sy, sz, st):
    B, G, C = gx.shape
    TN = types_t.shape[1]
    spec_g = pl.BlockSpec((1, G, C), lambda b: (b, 0, 0))
    spec_k = pl.BlockSpec((1, K, C), lambda b: (b, 0, 0))
    out = pl.pallas_call(
        functools.partial(_local_body, G=G, TN=TN),
        grid=(B,),
        in_specs=[
            spec_g, spec_g, spec_g,
            pl.BlockSpec((1, TN, G, C), lambda b: (b, 0, 0, 0)),
            spec_k, spec_k, spec_k, spec_k,
        ],
        out_specs=pl.BlockSpec((1, 1, 128), lambda b: (b, 0, 0)),
        out_shape=jax.ShapeDtypeStruct((B, 1, 128), jnp.float32),
    )(gx, gy, gz, types_t, sx, sy, sz, st)
    return jnp.sum(out, axis=(0, 1))


def kernel(center_coords, coords, types, target):
    B, C, _ = center_coords.shape
    G = coords.shape[2]
    T = target.shape[1]

    target_t = jnp.transpose(target, (0, 2, 1))[:, :3]
    idx = _topk_indices(center_coords, target_t)

    table = jnp.pad(target, ((0, 0), (0, 0), (0, 12))).reshape(B * T, 16)
    rows = _sc_gather(table, idx.reshape(-1))
    ge = rows.reshape(B, C, K, 16)
    sel = jnp.transpose(ge[..., :4], (0, 3, 2, 1))
    sx, sy, sz, st = sel[:, 0], sel[:, 1], sel[:, 2], sel[:, 3]

    gco = jnp.transpose(coords, (0, 3, 2, 1))
    gx, gy, gz = gco[:, 0], gco[:, 1], gco[:, 2]
    types_t = jnp.transpose(types, (0, 3, 2, 1))
    p = _local_losses(gx, gy, gz, types_t, sx, sy, sz, st)

    n_g = B * C * G
    n_k = B * C * K
    gen_type_loss = -p[2] / n_g
    target_type_loss = -p[4] / n_k
    emd_loss = p[0] / n_g + p[1] / n_k
    gen_type_correct = p[3] / n_g
    target_type_correct = p[5] / n_k
    loss = gen_type_loss + target_type_loss + emd_loss
    return (loss, gen_type_loss, target_type_loss, emd_loss,
            gen_type_correct, target_type_correct)


# speedup vs baseline: 22.0454x; 1.0352x over previous
import functools

import jax
import jax.numpy as jnp
from jax import lax
from jax.experimental import pallas as pl
from jax.experimental.pallas import tpu as pltpu
from jax.experimental.pallas import tpu_sc as plsc

K = 10


def _topk_body(c_ref, t_ref, o_ref, *, n_target):
    b = pl.program_id(0)
    cc = c_ref[0]
    tt = t_ref[0]
    d = (cc[:, 0:1] - tt[0:1, :]) ** 2
    d = d + (cc[:, 1:2] - tt[1:2, :]) ** 2
    d = d + (cc[:, 2:3] - tt[2:3, :]) ** 2
    iota = lax.broadcasted_iota(jnp.int32, d.shape, 1)
    key = (lax.bitcast_convert_type(d, jnp.int32) & jnp.int32(-4096)) | iota
    base = b * n_target
    for k in range(K):
        m = jnp.min(key, axis=1, keepdims=True)
        o_ref[0, :, k : k + 1] = (m & 4095) + base
        if k < K - 1:
            key = jnp.where(key == m, jnp.int32(0x7FFFFFFF), key)


def _topk_indices(center_coords, target_t):
    B, C, _ = center_coords.shape
    T = target_t.shape[-1]
    CB = 512
    nb = C // CB
    return pl.pallas_call(
        functools.partial(_topk_body, n_target=T),
        grid=(B, nb),
        in_specs=[
            pl.BlockSpec((1, CB, 3), lambda b, i: (b, i, 0)),
            pl.BlockSpec((1, 3, T), lambda b, i: (b, 0, 0)),
        ],
        out_specs=pl.BlockSpec((1, CB, K), lambda b, i: (b, i, 0)),
        out_shape=jax.ShapeDtypeStruct((B, C, K), jnp.int32),
    )(center_coords, target_t)


def _sc_gather(table, idx):
    N = idx.shape[0]
    info = plsc.get_sparse_core_info()
    nw = info.num_cores * info.num_subcores
    n_per = N // nw
    CH = 128
    nch = n_per // CH
    mesh = plsc.VectorSubcoreMesh(core_axis_name="c", subcore_axis_name="s")

    @functools.partial(
        pl.kernel,
        mesh=mesh,
        compiler_params=pltpu.CompilerParams(use_tc_tiling_on_sc=False),
        out_type=jax.ShapeDtypeStruct((N, 16), jnp.float32),
        scratch_types=[
            pltpu.VMEM((n_per,), jnp.int32),
            pltpu.VMEM((n_per, 16), jnp.float32),
            pltpu.SemaphoreType.DMA,
        ],
    )
    def gather_k(table_hbm, idx_hbm, out_hbm, idx_v, rows_v, sem):
        wid = lax.axis_index("s") * info.num_cores + lax.axis_index("c")
        base = wid * n_per
        pltpu.sync_copy(idx_hbm.at[pl.ds(base, n_per)], idx_v)
        copies = []
        for j in range(nch):
            copies.append(
                pltpu.async_copy(
                    table_hbm.at[idx_v.at[pl.ds(j * CH, CH)]],
                    rows_v.at[pl.ds(j * CH, CH)],
                    sem,
                )
            )
        for c in copies:
            c.wait()
        pltpu.sync_copy(rows_v, out_hbm.at[pl.ds(base, n_per)])

    return gather_k(table, idx)


def _local_body(gx_ref, gy_ref, gz_ref, tyT_ref, sx_ref, sy_ref, sz_ref, st_ref,
                o_ref, *, G, TN):
    gx = gx_ref[0]
    gy = gy_ref[0]
    gz = gz_ref[0]
    sx = sx_ref[0]
    sy = sy_ref[0]
    sz = sz_ref[0]
    sti = st_ref[0].astype(jnp.int32)
    iota_g = lax.broadcasted_iota(jnp.int32, gx.shape, 0)

    M = [tyT_ref[0, t] for t in range(TN)]
    mx = M[0]
    amax_t = jnp.zeros_like(iota_g)
    for t in range(1, TN):
        hi = M[t] > mx
        amax_t = jnp.where(hi, t, amax_t)
        mx = jnp.where(hi, M[t], mx)

    demd_gen = jnp.full(gx.shape, 3.4e38, jnp.float32)
    ks = jnp.zeros_like(iota_g)
    s_emd_tgt = jnp.float32(0.0)
    s_picked_tgt = jnp.float32(0.0)
    s_corr_tgt = jnp.float32(0.0)
    for k in range(K):
        dk = (gx - sx[k : k + 1]) ** 2
        dk = dk + (gy - sy[k : k + 1]) ** 2
        dk = dk + (gz - sz[k : k + 1]) ** 2
        lo = dk < demd_gen
        ks = jnp.where(lo, k, ks)
        demd_gen = jnp.where(lo, dk, demd_gen)
        mg = jnp.min(dk, axis=0, keepdims=True)
        s_emd_tgt = s_emd_tgt + jnp.sum(mg)
        gs_k = jnp.min(jnp.where(dk == mg, iota_g, G), axis=0, keepdims=True)
        ohg = iota_g == gs_k
        ttk = sti[k : k + 1]
        pick_cg = jnp.zeros_like(gx)
        for t in range(TN):
            pick_cg = pick_cg + jnp.where(ttk == t, M[t], 0.0)
        s_picked_tgt = s_picked_tgt + jnp.sum(jnp.where(ohg, pick_cg, 0.0))
        am_sel = jnp.sum(jnp.where(ohg, amax_t, 0), axis=0, keepdims=True)
        s_corr_tgt = s_corr_tgt + jnp.sum((am_sel == ttk).astype(jnp.float32))

    s_emd_gen = jnp.sum(demd_gen)
    tt_gen = jnp.zeros_like(iota_g)
    for k in range(K):
        tt_gen = tt_gen + jnp.where(ks == k, sti[k : k + 1], 0)
    picked_gen = jnp.zeros_like(gx)
    for t in range(TN):
        picked_gen = picked_gen + jnp.where(tt_gen == t, M[t], 0.0)
    s_picked_gen = jnp.sum(picked_gen)
    s_corr_gen = jnp.sum((amax_t == tt_gen).astype(jnp.float32))

    lane = lax.broadcasted_iota(jnp.int32, (1, 128), 1)
    vec = jnp.where(lane == 0, s_emd_gen, 0.0)
    vec = vec + jnp.where(lane == 1, s_emd_tgt, 0.0)
    vec = vec + jnp.where(lane == 2, s_picked_gen, 0.0)
    vec = vec + jnp.where(lane == 3, s_corr_gen, 0.0)
    vec = vec + jnp.where(lane == 4, s_picked_tgt, 0.0)
    vec = vec + jnp.where(lane == 5, s_corr_tgt, 0.0)
    o_ref[0] = vec


def _local_losses(gx, gy, gz, types_t, sx, sy, sz, st):
    B, G, C = gx.shape
    TN = types_t.shape[1]
    spec_g = pl.BlockSpec((1, G, C), lambda b: (b, 0, 0))
    spec_k = pl.BlockSpec((1, K, C), lambda b: (b, 0, 0))
    out = pl.pallas_call(
        functools.partial(_local_body, G=G, TN=TN),
        grid=(B,),
        in_specs=[
            spec_g, spec_g, spec_g,
            pl.BlockSpec((1, TN, G, C), lambda b: (b, 0, 0, 0)),
            spec_k, spec_k, spec_k, spec_k,
        ],
        out_specs=pl.BlockSpec((1, 1, 128), lambda b: (b, 0, 0)),
        out_shape=jax.ShapeDtypeStruct((B, 1, 128), jnp.float32),
    )(gx, gy, gz, types_t, sx, sy, sz, st)
    return jnp.sum(out, axis=(0, 1))


def kernel(center_coords, coords, types, target):
    B, C, _ = center_coords.shape
    G = coords.shape[2]
    T = target.shape[1]

    target_t = jnp.transpose(target, (0, 2, 1))[:, :3]
    idx = _topk_indices(center_coords, target_t)

    table = jnp.pad(target, ((0, 0), (0, 0), (0, 12))).reshape(B * T, 16)
    rows = _sc_gather(table, idx.reshape(-1))
    ge = rows.reshape(B, C, K, 16)
    sel = jnp.transpose(ge[..., :4], (0, 3, 2, 1))
    sx, sy, sz, st = sel[:, 0], sel[:, 1], sel[:, 2], sel[:, 3]

    gco = jnp.transpose(coords, (0, 3, 2, 1))
    gx, gy, gz = gco[:, 0], gco[:, 1], gco[:, 2]
    types_t = jnp.transpose(types, (0, 3, 2, 1))
    p = _local_losses(gx, gy, gz, types_t, sx, sy, sz, st)

    n_g = B * C * G
    n_k = B * C * K
    gen_type_loss = -p[2] / n_g
    target_type_loss = -p[4] / n_k
    emd_loss = p[0] / n_g + p[1] / n_k
    gen_type_correct = p[3] / n_g
    target_type_correct = p[5] / n_k
    loss = gen_type_loss + target_type_loss + emd_loss
    return (loss, gen_type_loss, target_type_loss, emd_loss,
            gen_type_correct, target_type_correct)
